# baseline (speedup 1.0000x reference)
import functools

import jax
import jax.numpy as jnp
import numpy as np
from jax.experimental import pallas as pl
from jax.experimental.pallas import tpu as pltpu

_N_CLASSES = 128
_N_BINS = 10
_BOUNDS = [np.float32(v) for v in np.linspace(0.0, 1.0, _N_BINS + 1)[:-1]]

_ROWS = 4096


def _hist_kernel(x_ref, t_ref, loss_ref, eacc_ref, *, n_steps, total):
    step = pl.program_id(0)

    x = x_ref[...]
    t = t_ref[...]
    m = jnp.max(x, axis=1, keepdims=True)
    e = jnp.exp(x - m)
    s = jnp.sum(e, axis=1, keepdims=True)
    r = 1.0 / s
    p = e * r

    cls = jax.lax.broadcasted_iota(jnp.int32, (_ROWS, _N_CLASSES), 1)
    it = t == cls
    gtb = it.astype(jnp.bfloat16)
    p_t = jnp.sum(jnp.where(it, p, 0.0), axis=1, keepdims=True)

    rmc = jnp.concatenate(
        [(p_t > float(u)).astype(jnp.bfloat16) for u in _BOUNDS]
        + [jnp.zeros((_ROWS, 16 - _N_BINS), jnp.bfloat16)], axis=1)

    a_cum = jax.lax.dot_general(
        rmc, gtb, (((0,), (0,)), ((), ())),
        preferred_element_type=jnp.float32)

    pb = p.astype(jnp.bfloat16)
    ones = jnp.ones((8, _ROWS), jnp.bfloat16)
    zero_b = jnp.bfloat16(0)
    c_rows = []
    sel0 = pb
    c_rows.append(jnp.dot(ones, sel0, preferred_element_type=jnp.float32)[0:1])
    for u in _BOUNDS[1:]:
        sel = jnp.where(p > u, pb, zero_b)
        c_rows.append(
            jnp.dot(ones, sel, preferred_element_type=jnp.float32)[0:1])
    c_cum = jnp.concatenate(
        c_rows + [jnp.zeros((16 - _N_BINS, _N_CLASSES), jnp.float32)], axis=0)

    upd = a_cum - c_cum

    @pl.when(step == 0)
    def _():
        eacc_ref[...] = upd

    @pl.when(step > 0)
    def _():
        eacc_ref[...] = eacc_ref[...] + upd

    @pl.when(step == n_steps - 1)
    def _():
        a = eacc_ref[...]
        e_cum = a[0:_N_BINS]
        e_next = jnp.concatenate(
            [a[1:_N_BINS], jnp.zeros((1, _N_CLASSES), jnp.float32)], axis=0)
        per_bin = e_cum - e_next
        loss_ref[0, 0] = jnp.sum(jnp.abs(per_bin)) / total


def kernel(output, target):
    batch, n_classes = output.shape
    n_steps = batch // _ROWS
    t2 = target.reshape(batch, 1)

    loss = pl.pallas_call(
        functools.partial(_hist_kernel, n_steps=n_steps,
                          total=float(batch * n_classes)),
        grid=(n_steps,),
        in_specs=[
            pl.BlockSpec((_ROWS, n_classes), lambda i: (i, 0)),
            pl.BlockSpec((_ROWS, 1), lambda i: (i, 0)),
        ],
        out_specs=pl.BlockSpec((1, 1), lambda i: (0, 0), memory_space=pltpu.SMEM),
        out_shape=jax.ShapeDtypeStruct((1, 1), jnp.float32),
        scratch_shapes=[pltpu.VMEM((16, _N_CLASSES), jnp.float32)],
    )(output, t2)
    return loss[0, 0]
